# baseline (speedup 1.0000x reference)
import functools

import jax
import jax.numpy as jnp
from jax import lax
from jax.experimental import pallas as pl
from jax.experimental.pallas import tpu as pltpu
from jax.experimental.pallas import tpu_sc as plsc

B, N, F = 16, 2048, 2048
L = 16
NV = N // L
PV = 7
K = 100
N_TILE = 512

_NEG = -3e38
_POS = 3e38


def _gemv_body(x_ref, w_ref, b_ref, s_ref):
    res = lax.dot_general(
        w_ref[...], x_ref[0],
        (((1,), (1,)), ((), ())),
        preferred_element_type=jnp.float32,
    )
    s_ref[0] = res + b_ref[0]


def _gemv(x, W1, b1):
    return pl.pallas_call(
        _gemv_body,
        grid=(B, N // N_TILE),
        in_specs=[
            pl.BlockSpec((1, N_TILE, F), lambda b, n: (b, n, 0)),
            pl.BlockSpec((1, F), lambda b, n: (0, 0)),
            pl.BlockSpec(memory_space=pltpu.SMEM),
        ],
        out_specs=pl.BlockSpec((1, 1, N_TILE), lambda b, n: (b, 0, n)),
        out_shape=jax.ShapeDtypeStruct((B, 1, N), jnp.float32),
    )(x, W1, b1)


def _merge_desc(a, b):
    rb = lax.rev(b, (0,))
    hi = jnp.maximum(a, rb)
    lo = jnp.minimum(a, rb)
    hi, _ = plsc.sort_key_val(hi, hi, descending=True)
    lo, _ = plsc.sort_key_val(lo, lo, descending=True)
    return hi, lo


def _merge_asc(a, b):
    rb = lax.rev(b, (0,))
    lo = jnp.minimum(a, rb)
    hi = jnp.maximum(a, rb)
    lo, _ = plsc.sort_key_val(lo, lo)
    hi, _ = plsc.sort_key_val(hi, hi)
    return lo, hi


@functools.partial(
    pl.kernel,
    out_type=(
        jax.ShapeDtypeStruct((B, L), jnp.float32),
        jax.ShapeDtypeStruct((B, L), jnp.float32),
    ),
    mesh=plsc.VectorSubcoreMesh(core_axis_name="c", subcore_axis_name="s"),
    compiler_params=pltpu.CompilerParams(needs_layout_passes=False),
    scratch_types=[
        pltpu.VMEM((N,), jnp.float32),
        pltpu.VMEM((2 * PV * L,), jnp.float32),
        pltpu.VMEM((L,), jnp.float32),
        pltpu.VMEM((L,), jnp.float32),
    ],
)
def _sc_topk(s_hbm, w2_hbm, bias_hbm, prob_hbm, hat_hbm,
             row_v, w2_v, bias_v, out_v):
    cid = lax.axis_index("c")
    sid = lax.axis_index("s")

    @pl.when(cid == 0)
    def _():
        pltpu.sync_copy(s_hbm.at[sid], row_v)
        pltpu.sync_copy(w2_hbm, w2_v)
        pltpu.sync_copy(bias_hbm, bias_v)

        init = ((jnp.full((L,), _NEG, jnp.float32),) * PV
                + (jnp.full((L,), _POS, jnp.float32),) * PV)

        def body(i, pools):
            v = row_v[pl.ds(i * L, L)]
            vd, _ = plsc.sort_key_val(v, v, descending=True)
            new = []
            carry = vd
            for k in range(PV):
                hi, carry = _merge_desc(pools[k], carry)
                new.append(hi)
            va, _ = plsc.sort_key_val(v, v)
            carry = va
            for k in range(PV):
                lo, carry = _merge_asc(pools[PV + k], carry)
                new.append(lo)
            return tuple(new)

        pools = lax.fori_loop(0, NV, body, init)

        acc = jnp.zeros((L,), jnp.float32)
        for k in range(2 * PV):
            acc = acc + pools[k] * w2_v[pl.ds(k * L, L)]
        total = jnp.sum(acc)

        logit = jnp.full((L,), total) + bias_v[...]
        prob = 1.0 / (1.0 + jnp.exp(-logit))
        out_v[...] = prob
        pltpu.sync_copy(out_v, prob_hbm.at[sid])
        out_v[...] = jnp.where(prob >= 0.5, 1.0, 0.0).astype(jnp.float32)
        pltpu.sync_copy(out_v, hat_hbm.at[sid])


def kernel(x, W1, b1, W2, b2):
    s = _gemv(x, W1, b1).reshape(B, N)
    zeros12 = jnp.zeros((PV * L - K,), jnp.float32)
    w2pad = jnp.concatenate([W2[0, :K], zeros12, W2[0, K:], zeros12])
    biasv = jnp.broadcast_to(b2.astype(jnp.float32), (L,))
    prob, hat = _sc_topk(s, w2pad, biasv)
    return (prob[:, :1], hat[:, :1])


# speedup vs baseline: 1.1381x; 1.1381x over previous
import functools

import jax
import jax.numpy as jnp
from jax import lax
from jax.experimental import pallas as pl
from jax.experimental.pallas import tpu as pltpu
from jax.experimental.pallas import tpu_sc as plsc

B, N, F = 16, 2048, 2048
L = 16
NV = N // L
PV = 7
K = 100
N_TILE = 1024

_NEG = -3e38
_POS = 3e38


def _gemv_body(x_ref, w_ref, b_ref, s_ref):
    res = lax.dot_general(
        w_ref[...], x_ref[...],
        (((1,), (1,)), ((), ())),
        preferred_element_type=jnp.float32,
    )
    s_ref[...] = res + b_ref[0]


def _gemv(x, W1, b1):
    xf = x.reshape(B * N, F)
    return pl.pallas_call(
        _gemv_body,
        grid=(B * N // N_TILE,),
        in_specs=[
            pl.BlockSpec((N_TILE, F), lambda n: (n, 0)),
            pl.BlockSpec((1, F), lambda n: (0, 0)),
            pl.BlockSpec(memory_space=pltpu.SMEM),
        ],
        out_specs=pl.BlockSpec((1, N_TILE), lambda n: (0, n)),
        out_shape=jax.ShapeDtypeStruct((1, B * N), jnp.float32),
    )(xf, W1, b1)


def _merge_desc(a, b):
    rb = lax.rev(b, (0,))
    hi = jnp.maximum(a, rb)
    lo = jnp.minimum(a, rb)
    hi, _ = plsc.sort_key_val(hi, hi, descending=True)
    lo, _ = plsc.sort_key_val(lo, lo, descending=True)
    return hi, lo


def _merge_asc(a, b):
    rb = lax.rev(b, (0,))
    lo = jnp.minimum(a, rb)
    hi = jnp.maximum(a, rb)
    lo, _ = plsc.sort_key_val(lo, lo)
    hi, _ = plsc.sort_key_val(hi, hi)
    return lo, hi


@functools.partial(
    pl.kernel,
    out_type=(
        jax.ShapeDtypeStruct((B, L), jnp.float32),
        jax.ShapeDtypeStruct((B, L), jnp.float32),
    ),
    mesh=plsc.VectorSubcoreMesh(core_axis_name="c", subcore_axis_name="s"),
    compiler_params=pltpu.CompilerParams(needs_layout_passes=False),
    scratch_types=[
        pltpu.VMEM((N,), jnp.float32),
        pltpu.VMEM((2 * PV * L,), jnp.float32),
        pltpu.VMEM((L,), jnp.float32),
        pltpu.VMEM((L,), jnp.float32),
    ],
)
def _sc_topk(s_hbm, w2_hbm, bias_hbm, prob_hbm, hat_hbm,
             row_v, w2_v, bias_v, out_v):
    cid = lax.axis_index("c")
    sid = lax.axis_index("s")

    @pl.when(cid == 0)
    def _():
        pltpu.sync_copy(s_hbm.at[sid], row_v)
        pltpu.sync_copy(w2_hbm, w2_v)
        pltpu.sync_copy(bias_hbm, bias_v)

        init = ((jnp.full((L,), _NEG, jnp.float32),) * PV
                + (jnp.full((L,), _POS, jnp.float32),) * PV)

        def body(i, pools):
            v = row_v[pl.ds(i * L, L)]
            vd, _ = plsc.sort_key_val(v, v, descending=True)
            new = []
            carry = vd
            for k in range(PV):
                hi, carry = _merge_desc(pools[k], carry)
                new.append(hi)
            va, _ = plsc.sort_key_val(v, v)
            carry = va
            for k in range(PV):
                lo, carry = _merge_asc(pools[PV + k], carry)
                new.append(lo)
            return tuple(new)

        pools = lax.fori_loop(0, NV, body, init)

        acc = jnp.zeros((L,), jnp.float32)
        for k in range(2 * PV):
            acc = acc + pools[k] * w2_v[pl.ds(k * L, L)]
        total = jnp.sum(acc)

        logit = jnp.full((L,), total) + bias_v[...]
        prob = 1.0 / (1.0 + jnp.exp(-logit))
        out_v[...] = prob
        pltpu.sync_copy(out_v, prob_hbm.at[sid])
        out_v[...] = jnp.where(prob >= 0.5, 1.0, 0.0).astype(jnp.float32)
        pltpu.sync_copy(out_v, hat_hbm.at[sid])


def kernel(x, W1, b1, W2, b2):
    s = _gemv(x, W1, b1).reshape(B, N)
    zeros12 = jnp.zeros((PV * L - K,), jnp.float32)
    w2pad = jnp.concatenate([W2[0, :K], zeros12, W2[0, K:], zeros12])
    biasv = jnp.broadcast_to(b2.astype(jnp.float32), (L,))
    prob, hat = _sc_topk(s, w2pad, biasv)
    return (prob[:, :1], hat[:, :1])
